# Initial kernel scaffold; baseline (speedup 1.0000x reference)
import jax
import jax.numpy as jnp
from jax.experimental import pallas as pl


def kernel(categorical_features, continuous_features, linear_table, bias, embedding_table, W_lin, b_lin, W_emb, b_emb):
    raise NotImplementedError("write your pallas kernel here")


# speedup vs baseline: 1.1028x; 1.1028x over previous
import functools

import jax
import jax.numpy as jnp
from jax import lax
from jax.experimental import pallas as pl
from jax.experimental.pallas import tpu as pltpu
from jax.experimental.pallas import tpu_sc as plsc

NC = 2
NS = 16
LANES = 16


def _sc_gather_fm(idx3, emb_table, lin_table, B, F, D, R):
    NW = NC * NS
    bpw = B // NW
    CH = bpw // R
    IPC = R * F
    G = D // LANES

    mesh = plsc.VectorSubcoreMesh(core_axis_name="c", subcore_axis_name="s")

    @functools.partial(
        pl.kernel,
        out_type=(
            jax.ShapeDtypeStruct((B, D), jnp.float32),
            jax.ShapeDtypeStruct((B, LANES), jnp.float32),
            jax.ShapeDtypeStruct((B, LANES), jnp.float32),
        ),
        mesh=mesh,
        compiler_params=pltpu.CompilerParams(use_tc_tiling_on_sc=False),
        scratch_types=[
            pltpu.VMEM((CH, IPC), jnp.int32),
            pltpu.VMEM((IPC, D), jnp.float32),
            pltpu.VMEM((IPC, LANES), jnp.float32),
            pltpu.VMEM((R, D), jnp.float32),
            pltpu.VMEM((R, LANES), jnp.float32),
            pltpu.VMEM((R, LANES), jnp.float32),
        ],
    )
    def body(idx_hbm, emb_hbm, lin_hbm, s_out, q_out, l_out,
             idx_v, emb_b, lin_b, s_c, q_c, l_c):
        wid = lax.axis_index("s") * NC + lax.axis_index("c")
        pltpu.sync_copy(idx_hbm.at[wid], idx_v)

        def chunk(j, carry):
            idx_row = idx_v.at[j]
            pltpu.sync_copy(emb_hbm.at[idx_row], emb_b)
            pltpu.sync_copy(lin_hbm.at[idx_row], lin_b)
            for r in range(R):
                q = jnp.zeros((LANES,), jnp.float32)
                for g in range(G):
                    acc = jnp.zeros((LANES,), jnp.float32)

                    def fbody(f, c):
                        a, qq = c
                        v = emb_b[r * F + f, pl.ds(g * LANES, LANES)]
                        return a + v, qq + v * v

                    acc, q = lax.fori_loop(0, F, fbody, (acc, q))
                    s_c[r, pl.ds(g * LANES, LANES)] = acc
                q_c[r, :] = q

                def lbody(f, c):
                    return c + lin_b[r * F + f, :]

                l_c[r, :] = lax.fori_loop(
                    0, F, lbody, jnp.zeros((LANES,), jnp.float32))
            row0 = wid * bpw + j * R
            pltpu.sync_copy(s_c, s_out.at[pl.ds(row0, R)])
            pltpu.sync_copy(q_c, q_out.at[pl.ds(row0, R)])
            pltpu.sync_copy(l_c, l_out.at[pl.ds(row0, R)])
            return carry

        lax.fori_loop(0, CH, chunk, 0)

    return body(idx3, emb_table, lin_table)


def _tc_combine(S, Q, lv, x, bias, W_lin, b_lin, W_emb, b_emb, B, D):
    def body(s_ref, q_ref, lv_ref, x_ref, bias_ref, wlin_ref, blin_ref,
             wemb_ref, bemb_ref, o_ref):
        xs = x_ref[...]
        ec = xs * wemb_ref[...] + bemb_ref[...]
        stot = s_ref[...] + ec
        qtot = (jnp.sum(q_ref[...], axis=1, keepdims=True)
                + jnp.sum(ec * ec, axis=1, keepdims=True))
        inter = 0.5 * (jnp.sum(stot * stot, axis=1, keepdims=True) - qtot)
        lin = (jnp.sum(lv_ref[...], axis=1, keepdims=True)
               + bias_ref[0, 0] + xs * wlin_ref[0, 0] + blin_ref[0, 0])
        o_ref[...] = lin + inter

    return pl.pallas_call(
        body,
        out_shape=jax.ShapeDtypeStruct((B, 1), jnp.float32),
    )(S, Q, lv, x, bias, W_lin, b_lin, W_emb, b_emb)


def kernel(categorical_features, continuous_features, linear_table, bias,
           embedding_table, W_lin, b_lin, W_emb, b_emb):
    B, F = categorical_features.shape
    V, D = embedding_table.shape
    NW = NC * NS
    R = 4
    CH = (B // NW) // R
    IPC = R * F

    idx3 = categorical_features.astype(jnp.int32).reshape(NW, CH, IPC)
    lin16 = jnp.pad(linear_table, ((0, 0), (0, LANES - 1)))
    S, Q, lv = _sc_gather_fm(idx3, embedding_table, lin16, B, F, D, R)
    out = _tc_combine(S, Q, lv, continuous_features,
                      bias.reshape(1, 1), W_lin.reshape(1, 1),
                      b_lin.reshape(1, 1), W_emb.reshape(1, D),
                      b_emb.reshape(1, D), B, D)
    return out[:, 0]


# speedup vs baseline: 1.3333x; 1.2090x over previous
import functools

import jax
import jax.numpy as jnp
from jax import lax
from jax.experimental import pallas as pl
from jax.experimental.pallas import tpu as pltpu
from jax.experimental.pallas import tpu_sc as plsc

NC = 2
NS = 16
LANES = 16


def _sc_gather_fm(idx3, emb_table, lin_table, B, F, D, R):
    NW = NC * NS
    bpw = B // NW
    CH = bpw // R
    IPC = R * F
    G = D // LANES

    mesh = plsc.VectorSubcoreMesh(core_axis_name="c", subcore_axis_name="s")

    @functools.partial(
        pl.kernel,
        out_type=(
            jax.ShapeDtypeStruct((B, D), jnp.float32),
            jax.ShapeDtypeStruct((B, LANES), jnp.float32),
            jax.ShapeDtypeStruct((B, LANES), jnp.float32),
        ),
        mesh=mesh,
        compiler_params=pltpu.CompilerParams(use_tc_tiling_on_sc=False),
        scratch_types=[
            pltpu.VMEM((CH, IPC), jnp.int32),
            pltpu.VMEM((2, IPC, D), jnp.float32),
            pltpu.VMEM((2, IPC, LANES), jnp.float32),
            pltpu.VMEM((bpw, D), jnp.float32),
            pltpu.VMEM((bpw, LANES), jnp.float32),
            pltpu.VMEM((bpw, LANES), jnp.float32),
            pltpu.SemaphoreType.DMA,
            pltpu.SemaphoreType.DMA,
        ],
    )
    def body(idx_hbm, emb_hbm, lin_hbm, s_out, q_out, l_out,
             idx_v, emb_b, lin_b, s_loc, q_loc, l_loc, sem0, sem1):
        wid = lax.axis_index("s") * NC + lax.axis_index("c")
        pltpu.sync_copy(idx_hbm.at[wid], idx_v)
        sems = (sem0, sem1)

        def issue(j, slot):
            row = idx_v.at[j]
            pltpu.async_copy(emb_hbm.at[row], emb_b.at[slot], sems[slot])
            pltpu.async_copy(lin_hbm.at[row], lin_b.at[slot], sems[slot])

        def drain(j, slot):
            row = idx_v.at[j]
            pltpu.make_async_copy(emb_hbm.at[row], emb_b.at[slot],
                                  sems[slot]).wait()
            pltpu.make_async_copy(lin_hbm.at[row], lin_b.at[slot],
                                  sems[slot]).wait()

        def compute(j, slot):
            for r in range(R):
                q = None
                for g in range(G):
                    acc = None
                    for f in range(F):
                        v = emb_b[slot, r * F + f, pl.ds(g * LANES, LANES)]
                        acc = v if acc is None else acc + v
                        q = v * v if q is None else q + v * v
                    s_loc[j * R + r, pl.ds(g * LANES, LANES)] = acc
                q_loc[j * R + r, :] = q
                lsum = None
                for f in range(F):
                    lv = lin_b[slot, r * F + f, :]
                    lsum = lv if lsum is None else lsum + lv
                l_loc[j * R + r, :] = lsum

        issue(0, 0)
        issue(1, 1)

        @pl.loop(0, CH, step=2)
        def _(j0):
            drain(j0, 0)
            compute(j0, 0)

            @pl.when(j0 + 2 < CH)
            def _():
                issue(j0 + 2, 0)

            drain(j0 + 1, 1)
            compute(j0 + 1, 1)

            @pl.when(j0 + 3 < CH)
            def _():
                issue(j0 + 3, 1)

        pltpu.sync_copy(s_loc, s_out.at[pl.ds(wid * bpw, bpw)])
        pltpu.sync_copy(q_loc, q_out.at[pl.ds(wid * bpw, bpw)])
        pltpu.sync_copy(l_loc, l_out.at[pl.ds(wid * bpw, bpw)])

    return body(idx3, emb_table, lin_table)


def _tc_combine(S, Q, lv, x, bias, W_lin, b_lin, W_emb, b_emb, B, D):
    def body(s_ref, q_ref, lv_ref, x_ref, bias_ref, wlin_ref, blin_ref,
             wemb_ref, bemb_ref, o_ref):
        xs = x_ref[...]
        ec = xs * wemb_ref[...] + bemb_ref[...]
        stot = s_ref[...] + ec
        qtot = (jnp.sum(q_ref[...], axis=1, keepdims=True)
                + jnp.sum(ec * ec, axis=1, keepdims=True))
        inter = 0.5 * (jnp.sum(stot * stot, axis=1, keepdims=True) - qtot)
        lin = (jnp.sum(lv_ref[...], axis=1, keepdims=True)
               + bias_ref[0, 0] + xs * wlin_ref[0, 0] + blin_ref[0, 0])
        o_ref[...] = lin + inter

    return pl.pallas_call(
        body,
        out_shape=jax.ShapeDtypeStruct((B, 1), jnp.float32),
    )(S, Q, lv, x, bias, W_lin, b_lin, W_emb, b_emb)


def kernel(categorical_features, continuous_features, linear_table, bias,
           embedding_table, W_lin, b_lin, W_emb, b_emb):
    B, F = categorical_features.shape
    V, D = embedding_table.shape
    NW = NC * NS
    R = 4
    CH = (B // NW) // R
    IPC = R * F

    idx3 = categorical_features.astype(jnp.int32).reshape(NW, CH, IPC)
    lin16 = jnp.pad(linear_table, ((0, 0), (0, LANES - 1)))
    S, Q, lv = _sc_gather_fm(idx3, embedding_table, lin16, B, F, D, R)
    out = _tc_combine(S, Q, lv, continuous_features,
                      bias.reshape(1, 1), W_lin.reshape(1, 1),
                      b_lin.reshape(1, 1), W_emb.reshape(1, D),
                      b_emb.reshape(1, D), B, D)
    return out[:, 0]


# speedup vs baseline: 2.1345x; 1.6009x over previous
import functools

import jax
import jax.numpy as jnp
from jax import lax
from jax.experimental import pallas as pl
from jax.experimental.pallas import tpu as pltpu
from jax.experimental.pallas import tpu_sc as plsc

NC = 2
NS = 16
LANES = 16


def _sc_gather_fm(idx3, emb2d, lin1d, B, F, D, R):
    NW = NC * NS
    bpw = B // NW
    CH = bpw // R
    IPC = R * F
    G = D // LANES
    V = lin1d.shape[0]
    NFULL = IPC // LANES

    mesh = plsc.VectorSubcoreMesh(core_axis_name="c", subcore_axis_name="s")

    @functools.partial(
        pl.kernel,
        out_type=(
            jax.ShapeDtypeStruct((B, D), jnp.float32),
            jax.ShapeDtypeStruct((B, LANES), jnp.float32),
        ),
        mesh=mesh,
        compiler_params=pltpu.CompilerParams(use_tc_tiling_on_sc=False,
                                             needs_layout_passes=False),
        scratch_types=[
            pltpu.VMEM((CH, IPC), jnp.int32),
            pltpu.VMEM((2, IPC, D), jnp.float32),
            pltpu.VMEM((V,), jnp.float32),
            pltpu.VMEM((bpw, D), jnp.float32),
            pltpu.VMEM((bpw, LANES), jnp.float32),
            pltpu.SemaphoreType.DMA,
            pltpu.SemaphoreType.DMA,
            pltpu.SemaphoreType.DMA,
        ],
    )
    def body(idx_hbm, emb_hbm, lin_hbm, s_out, q_out,
             idx_v, emb_b, lin_v, s_loc, q_loc, sem0, sem1, seml):
        wid = lax.axis_index("s") * NC + lax.axis_index("c")
        pltpu.async_copy(lin_hbm, lin_v, seml)
        pltpu.sync_copy(idx_hbm.at[wid], idx_v)
        sems = (sem0, sem1)

        def issue(j, slot):
            pltpu.async_copy(emb_hbm.at[idx_v.at[j]], emb_b.at[slot],
                             sems[slot])

        def drain(j, slot):
            pltpu.make_async_copy(emb_hbm.at[idx_v.at[j]], emb_b.at[slot],
                                  sems[slot]).wait()

        def compute(j, slot):
            for r in range(R):
                q = None
                for g in range(G):
                    acc = None
                    for f in range(F):
                        v = emb_b[slot, f * R + r, pl.ds(g * LANES, LANES)]
                        acc = v if acc is None else acc + v
                        q = v * v if q is None else q + v * v
                    s_loc[j * R + r, pl.ds(g * LANES, LANES)] = acc
                q_loc[j * R + r, :] = q

        issue(0, 0)
        issue(1, 1)

        @pl.loop(0, CH, step=2)
        def _(j0):
            drain(j0, 0)
            compute(j0, 0)

            @pl.when(j0 + 2 < CH)
            def _():
                issue(j0 + 2, 0)

            drain(j0 + 1, 1)
            compute(j0 + 1, 1)

            @pl.when(j0 + 3 < CH)
            def _():
                issue(j0 + 3, 1)

        pltpu.sync_copy(s_loc, s_out.at[pl.ds(wid * bpw, bpw)])

        pltpu.make_async_copy(lin_hbm, lin_v, seml).wait()
        lanes = jax.lax.broadcasted_iota(jnp.int32, (LANES,), 0)
        tail_keep = lanes >= (NFULL * LANES - (IPC - LANES))
        zeros = jnp.zeros((LANES,), jnp.float32)

        @pl.loop(0, CH)
        def _(j):
            lsum = None
            for m in range(NFULL):
                iv = idx_v[j, pl.ds(m * LANES, LANES)]
                vals = plsc.load_gather(lin_v, [iv])
                lsum = vals if lsum is None else lsum + vals
            if NFULL * LANES < IPC:
                iv = idx_v[j, pl.ds(IPC - LANES, LANES)]
                vals = plsc.load_gather(lin_v, [iv])
                lsum = lsum + jnp.where(tail_keep, vals, zeros)
            for r in range(R):
                contrib = jnp.where(lanes % R == r, lsum, zeros)
                q_loc[j * R + r, :] = q_loc[j * R + r, :] - 2.0 * contrib

        pltpu.sync_copy(q_loc, q_out.at[pl.ds(wid * bpw, bpw)])

    return body(idx3, emb2d, lin1d)


def _tc_combine(S, Qm, x, bias, W_lin, b_lin, W_emb, b_emb, B, D):
    def body(s_ref, q_ref, x_ref, bias_ref, wlin_ref, blin_ref,
             wemb_ref, bemb_ref, o_ref):
        xs = x_ref[...]
        ec = xs * wemb_ref[...] + bemb_ref[...]
        stot = s_ref[...] + ec
        qm = (jnp.sum(q_ref[...], axis=1, keepdims=True)
              + jnp.sum(ec * ec, axis=1, keepdims=True))
        inter = 0.5 * (jnp.sum(stot * stot, axis=1, keepdims=True) - qm)
        o_ref[...] = (inter + bias_ref[0, 0] + xs * wlin_ref[0, 0]
                      + blin_ref[0, 0])

    return pl.pallas_call(
        body,
        out_shape=jax.ShapeDtypeStruct((B, 1), jnp.float32),
    )(S, Qm, x, bias, W_lin, b_lin, W_emb, b_emb)


def kernel(categorical_features, continuous_features, linear_table, bias,
           embedding_table, W_lin, b_lin, W_emb, b_emb):
    B, F = categorical_features.shape
    V, D = embedding_table.shape
    NW = NC * NS
    R = 4
    CH = (B // NW) // R

    idx3 = (categorical_features.astype(jnp.int32)
            .reshape(NW, CH, R, F).transpose(0, 1, 3, 2)
            .reshape(NW, CH, F * R))
    emb2d = lax.optimization_barrier(embedding_table.reshape(-1)).reshape(V, D)
    lin1d = linear_table.reshape(-1)
    S, Qm = _sc_gather_fm(idx3, emb2d, lin1d, B, F, D, R)
    out = _tc_combine(S, Qm, continuous_features,
                      bias.reshape(1, 1), W_lin.reshape(1, 1),
                      b_lin.reshape(1, 1), W_emb.reshape(1, D),
                      b_emb.reshape(1, D), B, D)
    return out[:, 0]


# speedup vs baseline: 2.1864x; 1.0243x over previous
import functools

import jax
import jax.numpy as jnp
from jax import lax
from jax.experimental import pallas as pl
from jax.experimental.pallas import tpu as pltpu
from jax.experimental.pallas import tpu_sc as plsc

NC = 2
NS = 16
LANES = 16


def _sc_fm(idx3, emb2d, lin1d, x1d, combo, B, F, D, R):
    NW = NC * NS
    bpw = B // NW
    CH = bpw // R
    IPC = R * F
    G = D // LANES
    V = lin1d.shape[0]
    NFULL = IPC // LANES
    NBLK = bpw // LANES
    CSZ = combo.shape[0]

    mesh = plsc.VectorSubcoreMesh(core_axis_name="c", subcore_axis_name="s")

    @functools.partial(
        pl.kernel,
        out_type=jax.ShapeDtypeStruct((B,), jnp.float32),
        mesh=mesh,
        compiler_params=pltpu.CompilerParams(use_tc_tiling_on_sc=False,
                                             needs_layout_passes=False),
        scratch_types=[
            pltpu.VMEM((CH, IPC), jnp.int32),
            pltpu.VMEM((2, IPC, D), jnp.float32),
            pltpu.VMEM((V,), jnp.float32),
            pltpu.VMEM((bpw * D,), jnp.float32),
            pltpu.VMEM((bpw * LANES,), jnp.float32),
            pltpu.VMEM((bpw,), jnp.float32),
            pltpu.VMEM((CSZ,), jnp.float32),
            pltpu.VMEM((bpw,), jnp.float32),
            pltpu.SemaphoreType.DMA,
            pltpu.SemaphoreType.DMA,
            pltpu.SemaphoreType.DMA,
        ],
    )
    def body(idx_hbm, emb_hbm, lin_hbm, x_hbm, combo_hbm, out_hbm,
             idx_v, emb_b, lin_v, s_loc, q_loc, x_v, combo_v, out_v,
             sem0, sem1, seml):
        wid = lax.axis_index("s") * NC + lax.axis_index("c")
        pltpu.async_copy(lin_hbm, lin_v, seml)
        pltpu.sync_copy(idx_hbm.at[wid], idx_v)
        pltpu.sync_copy(x_hbm.at[pl.ds(wid * bpw, bpw)], x_v)
        pltpu.sync_copy(combo_hbm, combo_v)
        sems = (sem0, sem1)

        def issue(j, slot):
            pltpu.async_copy(emb_hbm.at[idx_v.at[j]], emb_b.at[slot],
                             sems[slot])

        def drain(j, slot):
            pltpu.make_async_copy(emb_hbm.at[idx_v.at[j]], emb_b.at[slot],
                                  sems[slot]).wait()

        def compute(j, slot):
            for r in range(R):
                q = None
                for g in range(G):
                    acc = None
                    for f in range(F):
                        v = emb_b[slot, f * R + r, pl.ds(g * LANES, LANES)]
                        acc = v if acc is None else acc + v
                        q = v * v if q is None else q + v * v
                    s_loc[pl.ds((j * R + r) * D + g * LANES, LANES)] = acc
                q_loc[pl.ds((j * R + r) * LANES, LANES)] = q

        issue(0, 0)
        issue(1, 1)

        @pl.loop(0, CH, step=2)
        def _(j0):
            drain(j0, 0)
            compute(j0, 0)

            @pl.when(j0 + 2 < CH)
            def _():
                issue(j0 + 2, 0)

            drain(j0 + 1, 1)
            compute(j0 + 1, 1)

            @pl.when(j0 + 3 < CH)
            def _():
                issue(j0 + 3, 1)

        pltpu.make_async_copy(lin_hbm, lin_v, seml).wait()
        lanes = lax.broadcasted_iota(jnp.int32, (LANES,), 0)
        tail_keep = lanes >= (NFULL * LANES - (IPC - LANES))
        zeros = jnp.zeros((LANES,), jnp.float32)

        @pl.loop(0, CH)
        def _(j):
            lsum = None
            for m in range(NFULL):
                iv = idx_v[j, pl.ds(m * LANES, LANES)]
                vals = plsc.load_gather(lin_v, [iv])
                lsum = vals if lsum is None else lsum + vals
            if NFULL * LANES < IPC:
                iv = idx_v[j, pl.ds(IPC - LANES, LANES)]
                vals = plsc.load_gather(lin_v, [iv])
                lsum = lsum + jnp.where(tail_keep, vals, zeros)
            for r in range(R):
                contrib = jnp.where(lanes % R == r, lsum, zeros)
                base = (j * R + r) * LANES
                q_loc[pl.ds(base, LANES)] = (
                    q_loc[pl.ds(base, LANES)] - 2.0 * contrib)

        scal = combo_v[pl.ds(2 * D, LANES)]
        w_lin = scal[0]
        b_lin = scal[1]
        bias = scal[2]
        wv = [combo_v[pl.ds(g * LANES, LANES)] for g in range(G)]
        cv = [combo_v[pl.ds(D + g * LANES, LANES)] for g in range(G)]
        for blk in range(NBLK):
            xv = x_v[pl.ds(blk * LANES, LANES)]
            acc = zeros
            ecsq = zeros
            for d in range(D):
                sd = plsc.load_gather(
                    s_loc, [lanes * D + (blk * LANES * D + d)])
                ec = xv * wv[d // LANES][d % LANES] + cv[d // LANES][d % LANES]
                t = sd + ec
                acc = acc + t * t
                ecsq = ecsq + ec * ec
            qsum = zeros
            for k in range(LANES):
                qk = plsc.load_gather(
                    q_loc, [lanes * LANES + (blk * LANES * LANES + k)])
                qsum = qsum + qk
            out_v[pl.ds(blk * LANES, LANES)] = (
                0.5 * (acc - qsum - ecsq) + bias + xv * w_lin + b_lin)

        pltpu.sync_copy(out_v, out_hbm.at[pl.ds(wid * bpw, bpw)])

    return body(idx3, emb2d, lin1d, x1d, combo)


def kernel(categorical_features, continuous_features, linear_table, bias,
           embedding_table, W_lin, b_lin, W_emb, b_emb):
    B, F = categorical_features.shape
    V, D = embedding_table.shape
    NW = NC * NS
    R = 4
    CH = (B // NW) // R

    idx3 = (categorical_features.astype(jnp.int32)
            .reshape(NW, CH, R, F).transpose(0, 1, 3, 2)
            .reshape(NW, CH, F * R))
    emb2d = lax.optimization_barrier(embedding_table.reshape(-1)).reshape(V, D)
    lin1d = linear_table.reshape(-1)
    x1d = continuous_features.reshape(-1)
    combo = jnp.concatenate([
        W_emb.reshape(-1), b_emb.reshape(-1), W_lin.reshape(-1),
        b_lin.reshape(-1), bias.reshape(-1),
        jnp.zeros((2 * D + 16 - (2 * D + 3),), jnp.float32),
    ])
    return _sc_fm(idx3, emb2d, lin1d, x1d, combo, B, F, D, R)
